# baseline (speedup 1.0000x reference)
import functools

import jax
import jax.numpy as jnp
from jax import lax
from jax.experimental import pallas as pl
from jax.experimental.pallas import tpu as pltpu
from jax.experimental.pallas import tpu_sc as plsc

E = 64
C = 8
K = 16
L = 16
NC = 2
NS = 16
NW = NC * NS


def _build_sc_kernel(B: int):
  R = B // NW
  mesh = plsc.VectorSubcoreMesh(core_axis_name="c", subcore_axis_name="s")

  @functools.partial(
      pl.kernel,
      out_type=(
          jax.ShapeDtypeStruct((B, K), jnp.int32),
          jax.ShapeDtypeStruct((B, K), jnp.float32),
      ),
      mesh=mesh,
      compiler_params=pltpu.CompilerParams(
          needs_layout_passes=False, use_tc_tiling_on_sc=False),
      scratch_types=[
          pltpu.VMEM((E, E), jnp.float32),
          pltpu.VMEM((R + 2, C), jnp.int32),
          pltpu.VMEM((R + 2, C), jnp.float32),
          pltpu.VMEM((L,), jnp.int32),
          pltpu.VMEM((L,), jnp.float32),
          pltpu.VMEM((R, K), jnp.int32),
          pltpu.VMEM((R, K), jnp.float32),
          pltpu.VMEM((2, 5, L), jnp.float32),
          pltpu.VMEM((2, L), jnp.int32),
          pltpu.SemaphoreType.DMA,
      ],
  )
  def sc_kernel(scores_hbm, cooc_hbm, ids_hbm, dvi_hbm, dvf_hbm,
                oi_hbm, os_hbm,
                cooc_v, ids_v, sc_v, di_v, df_v, oi_v, os_v, pf_v, pi_v,
                dsem):
    wid = lax.axis_index("s") * NC + lax.axis_index("c")
    base = wid * R
    pltpu.sync_copy(cooc_hbm, cooc_v)
    pltpu.sync_copy(ids_hbm.at[pl.ds(base, R)], ids_v.at[pl.ds(0, R)])
    pltpu.sync_copy(scores_hbm.at[pl.ds(base, R)], sc_v.at[pl.ds(0, R)])
    pltpu.sync_copy(dvi_hbm, di_v)
    pltpu.sync_copy(dvf_hbm, df_v)

    lane = lax.iota(jnp.int32, L)
    mask_lo = lane < C
    neg_inf = jnp.full((L,), -jnp.inf, jnp.float32)
    vals = [lane + j * L for j in range(E // L)]
    col8 = lane & (C - 1)
    ocol = col8 + C
    splats = [jnp.full((L,), i, jnp.int32) for i in range(C)]
    di = di_v[...]
    df = df_v[...]

    def accum(r):
      rvec = jnp.full((L,), 0, jnp.int32) + r
      prods = [[] for _ in range(E // L)]
      masks = [[] for _ in range(E // L)]
      for i in range(C):
        idv = plsc.load_gather(ids_v, [rvec, splats[i]])
        sv = plsc.load_gather(sc_v, [rvec, splats[i]])
        for j in range(E // L):
          rowj = plsc.load_gather(cooc_v, [idv, vals[j]])
          prods[j].append(sv * rowj)
          masks[j].append(vals[j] == idv)

      def tree(xs, op):
        while len(xs) > 1:
          xs = [op(xs[k], xs[k + 1]) for k in range(0, len(xs) - 1, 2)] + (
              [xs[-1]] if len(xs) & 1 else [])
        return xs[0]

      accs = tuple(
          jnp.where(tree(masks[j], jnp.logical_or), neg_inf,
                    tree(prods[j], jnp.add))
          for j in range(E // L))
      ids16 = plsc.load_gather(ids_v, [rvec, col8])
      s16 = plsc.load_gather(sc_v, [rvec, col8])
      return accs + (ids16, s16)

    def level1(state):
      a0, a1, a2, a3, ids16, s16 = state
      s0 = plsc.sort_key_val(a0, vals[0], descending=True)
      s1 = plsc.sort_key_val(a1, vals[1], descending=False)
      s2 = plsc.sort_key_val(a2, vals[2], descending=True)
      s3 = plsc.sort_key_val(a3, vals[3], descending=False)
      return s0, s1, s2, s3, ids16, s16

    def finish(r, state):
      (s0k, s0v), (s1k, s1v), (s2k, s2v), (s3k, s3v), ids16, s16 = state
      t01k, t01v = plsc.sort_key_val(jnp.where(mask_lo, s0k, s1k),
                                     jnp.where(mask_lo, s0v, s1v),
                                     descending=True)
      t23k, t23v = plsc.sort_key_val(jnp.where(mask_lo, s2k, s3k),
                                     jnp.where(mask_lo, s2v, s3v),
                                     descending=False)
      fk, fv = plsc.sort_key_val(jnp.where(mask_lo, t01k, t23k),
                                 jnp.where(mask_lo, t01v, t23v),
                                 descending=True)
      rvec = jnp.full((L,), 0, jnp.int32) + r
      oi_v[r] = ids16 + di
      os_v[r] = s16 + df
      plsc.store_scatter(oi_v, [rvec, ocol], fv + di, mask=mask_lo)
      plsc.store_scatter(os_v, [rvec, ocol], fk + df, mask=mask_lo)

    def stash(slot, st):
      a0, a1, a2, a3, ids16, s16 = st
      pf_v[slot, 0] = a0
      pf_v[slot, 1] = a1
      pf_v[slot, 2] = a2
      pf_v[slot, 3] = a3
      pf_v[slot, 4] = s16
      pi_v[slot] = ids16

    def unstash(slot):
      return (pf_v[slot, 0], pf_v[slot, 1], pf_v[slot, 2], pf_v[slot, 3],
              pi_v[slot], pf_v[slot, 4])

    stash(1, accum(0))

    def row_body(r, carry):
      sorted1 = level1(unstash((r + 1) & 1))
      stash(r & 1, accum(jnp.minimum(r, R - 1)))
      finish(jnp.maximum(r - 1, 0), sorted1)
      return carry

    lax.fori_loop(0, R + 1, row_body, 0)
    pltpu.sync_copy(oi_v, oi_hbm.at[pl.ds(base, R)])
    pltpu.sync_copy(os_v, os_hbm.at[pl.ds(base, R)])

  return sc_kernel


@functools.cache
def _get_sc_kernel(B: int):
  return _build_sc_kernel(B)


def kernel(candidate_scores, cooccurrence, candidate_ids, target_size):
  B, _ = candidate_ids.shape
  delta_i = jnp.asarray(target_size, jnp.int32) - K
  dvi = jnp.full((L,), delta_i, jnp.int32)
  dvf = jnp.full((L,), delta_i.astype(jnp.float32), jnp.float32)
  n_chunks = 8
  h = B // n_chunks
  f = _get_sc_kernel(h)
  outs = [f(candidate_scores[c * h:(c + 1) * h], cooccurrence,
            candidate_ids[c * h:(c + 1) * h], dvi, dvf)
          for c in range(n_chunks)]
  return (jnp.concatenate([o[0] for o in outs], axis=0),
          jnp.concatenate([o[1] for o in outs], axis=0))


# speedup vs baseline: 1.2037x; 1.2037x over previous
import functools

import jax
import jax.numpy as jnp
from jax import lax
from jax.experimental import pallas as pl
from jax.experimental.pallas import tpu as pltpu
from jax.experimental.pallas import tpu_sc as plsc

E = 64
C = 8
K = 16
L = 16
NC = 2
NS = 16
NW = NC * NS


def _build_sc_kernel(B: int):
  R = B // NW
  mesh = plsc.VectorSubcoreMesh(core_axis_name="c", subcore_axis_name="s")

  @functools.partial(
      pl.kernel,
      out_type=(
          jax.ShapeDtypeStruct((B, K), jnp.int32),
          jax.ShapeDtypeStruct((B, K), jnp.float32),
      ),
      mesh=mesh,
      compiler_params=pltpu.CompilerParams(
          needs_layout_passes=False, use_tc_tiling_on_sc=False),
      scratch_types=[
          pltpu.VMEM((E, E), jnp.float32),
          pltpu.VMEM((R + 2, C), jnp.int32),
          pltpu.VMEM((R + 2, C), jnp.float32),
          pltpu.VMEM((L,), jnp.int32),
          pltpu.VMEM((L,), jnp.float32),
          pltpu.VMEM((R, K), jnp.int32),
          pltpu.VMEM((R, K), jnp.float32),
          pltpu.VMEM((2, 5, L), jnp.float32),
          pltpu.VMEM((2, L), jnp.int32),
          pltpu.SemaphoreType.DMA,
      ],
  )
  def sc_kernel(scores_hbm, cooc_hbm, ids_hbm, dvi_hbm, dvf_hbm,
                oi_hbm, os_hbm,
                cooc_v, ids_v, sc_v, di_v, df_v, oi_v, os_v, pf_v, pi_v,
                dsem):
    wid = lax.axis_index("s") * NC + lax.axis_index("c")
    base = wid * R
    pltpu.sync_copy(cooc_hbm, cooc_v)
    pltpu.sync_copy(ids_hbm.at[pl.ds(base, R)], ids_v.at[pl.ds(0, R)])
    pltpu.sync_copy(scores_hbm.at[pl.ds(base, R)], sc_v.at[pl.ds(0, R)])
    pltpu.sync_copy(dvi_hbm, di_v)
    pltpu.sync_copy(dvf_hbm, df_v)

    lane = lax.iota(jnp.int32, L)
    mask_lo = lane < C
    neg_inf = jnp.full((L,), -jnp.inf, jnp.float32)
    vals = [lane + j * L for j in range(E // L)]
    col8 = lane & (C - 1)
    ocol = col8 + C
    splats = [jnp.full((L,), i, jnp.int32) for i in range(C)]
    di = di_v[...]
    df = df_v[...]

    def accum(r):
      rvec = jnp.full((L,), 0, jnp.int32) + r
      prods = [[] for _ in range(E // L)]
      masks = [[] for _ in range(E // L)]
      for i in range(C):
        idv = plsc.load_gather(ids_v, [rvec, splats[i]])
        sv = plsc.load_gather(sc_v, [rvec, splats[i]])
        for j in range(E // L):
          rowj = plsc.load_gather(cooc_v, [idv, vals[j]])
          prods[j].append(sv * rowj)
          masks[j].append(vals[j] == idv)

      def tree(xs, op):
        while len(xs) > 1:
          xs = [op(xs[k], xs[k + 1]) for k in range(0, len(xs) - 1, 2)] + (
              [xs[-1]] if len(xs) & 1 else [])
        return xs[0]

      accs = tuple(
          jnp.where(tree(masks[j], jnp.logical_or), neg_inf,
                    tree(prods[j], jnp.add))
          for j in range(E // L))
      ids16 = plsc.load_gather(ids_v, [rvec, col8])
      s16 = plsc.load_gather(sc_v, [rvec, col8])
      return accs + (ids16, s16)

    def level1(state):
      a0, a1, a2, a3, ids16, s16 = state
      s0 = plsc.sort_key_val(a0, vals[0], descending=True)
      s1 = plsc.sort_key_val(a1, vals[1], descending=False)
      s2 = plsc.sort_key_val(a2, vals[2], descending=True)
      s3 = plsc.sort_key_val(a3, vals[3], descending=False)
      return s0, s1, s2, s3, ids16, s16

    def finish(r, state):
      (s0k, s0v), (s1k, s1v), (s2k, s2v), (s3k, s3v), ids16, s16 = state
      t01k, t01v = plsc.sort_key_val(jnp.where(mask_lo, s0k, s1k),
                                     jnp.where(mask_lo, s0v, s1v),
                                     descending=True)
      t23k, t23v = plsc.sort_key_val(jnp.where(mask_lo, s2k, s3k),
                                     jnp.where(mask_lo, s2v, s3v),
                                     descending=False)
      fk, fv = plsc.sort_key_val(jnp.where(mask_lo, t01k, t23k),
                                 jnp.where(mask_lo, t01v, t23v),
                                 descending=True)
      rvec = jnp.full((L,), 0, jnp.int32) + r
      oi_v[r] = ids16 + di
      os_v[r] = s16 + df
      plsc.store_scatter(oi_v, [rvec, ocol], fv + di, mask=mask_lo)
      plsc.store_scatter(os_v, [rvec, ocol], fk + df, mask=mask_lo)

    def stash(slot, st):
      a0, a1, a2, a3, ids16, s16 = st
      pf_v[slot, 0] = a0
      pf_v[slot, 1] = a1
      pf_v[slot, 2] = a2
      pf_v[slot, 3] = a3
      pf_v[slot, 4] = s16
      pi_v[slot] = ids16

    def unstash(slot):
      return (pf_v[slot, 0], pf_v[slot, 1], pf_v[slot, 2], pf_v[slot, 3],
              pi_v[slot], pf_v[slot, 4])

    stash(1, accum(0))

    def row_body(r, carry):
      sorted1 = level1(unstash((r + 1) & 1))
      stash(r & 1, accum(jnp.minimum(r, R - 1)))
      finish(jnp.maximum(r - 1, 0), sorted1)
      return carry

    lax.fori_loop(0, R + 1, row_body, 0)
    pltpu.sync_copy(oi_v, oi_hbm.at[pl.ds(base, R)])
    pltpu.sync_copy(os_v, os_hbm.at[pl.ds(base, R)])

  return sc_kernel


@functools.cache
def _get_sc_kernel(B: int):
  return _build_sc_kernel(B)


def kernel(candidate_scores, cooccurrence, candidate_ids, target_size):
  B, _ = candidate_ids.shape
  delta_i = jnp.asarray(target_size, jnp.int32) - K
  dvi = jnp.full((L,), delta_i, jnp.int32)
  dvf = jnp.full((L,), delta_i.astype(jnp.float32), jnp.float32)
  n_chunks = 4
  h = B // n_chunks
  f = _get_sc_kernel(h)
  outs = [f(candidate_scores[c * h:(c + 1) * h], cooccurrence,
            candidate_ids[c * h:(c + 1) * h], dvi, dvf)
          for c in range(n_chunks)]
  return (jnp.concatenate([o[0] for o in outs], axis=0),
          jnp.concatenate([o[1] for o in outs], axis=0))


# speedup vs baseline: 1.2040x; 1.0002x over previous
import functools

import jax
import jax.numpy as jnp
from jax import lax
from jax.experimental import pallas as pl
from jax.experimental.pallas import tpu as pltpu
from jax.experimental.pallas import tpu_sc as plsc

E = 64
C = 8
K = 16
L = 16
NC = 2
NS = 16
NW = NC * NS


def _build_sc_kernel(B: int):
  R = B // NW
  mesh = plsc.VectorSubcoreMesh(core_axis_name="c", subcore_axis_name="s")

  @functools.partial(
      pl.kernel,
      out_type=(
          jax.ShapeDtypeStruct((B, K), jnp.int32),
          jax.ShapeDtypeStruct((B, K), jnp.float32),
      ),
      mesh=mesh,
      compiler_params=pltpu.CompilerParams(
          needs_layout_passes=False, use_tc_tiling_on_sc=False),
      scratch_types=[
          pltpu.VMEM((E, E), jnp.float32),
          pltpu.VMEM((R, C), jnp.int32),
          pltpu.VMEM((R, C), jnp.float32),
          pltpu.VMEM((L,), jnp.int32),
          pltpu.VMEM((L,), jnp.float32),
          pltpu.VMEM((R, K), jnp.int32),
          pltpu.VMEM((R, K), jnp.float32),
          pltpu.VMEM((2, 5, L), jnp.float32),
          pltpu.VMEM((2, L), jnp.int32),
          pltpu.SemaphoreType.DMA,
      ],
  )
  def sc_kernel(scores_hbm, cooc_hbm, ids_hbm, dvi_hbm, dvf_hbm,
                oi_hbm, os_hbm,
                cooc_v, ids_v, sc_v, di_v, df_v, oi_v, os_v, pf_v, pi_v,
                dsem):
    wid = lax.axis_index("s") * NC + lax.axis_index("c")
    base = wid * R
    pltpu.sync_copy(cooc_hbm, cooc_v)
    pltpu.sync_copy(ids_hbm.at[pl.ds(base, R)], ids_v)
    pltpu.sync_copy(scores_hbm.at[pl.ds(base, R)], sc_v)
    pltpu.sync_copy(dvi_hbm, di_v)
    pltpu.sync_copy(dvf_hbm, df_v)

    lane = lax.iota(jnp.int32, L)
    mask_lo = lane < C
    neg_inf = jnp.full((L,), -jnp.inf, jnp.float32)
    vals = [lane + j * L for j in range(E // L)]
    col8 = lane & (C - 1)
    ocol = col8 + C
    splats = [jnp.full((L,), i, jnp.int32) for i in range(C)]
    di = di_v[...]
    df = df_v[...]

    def accum(r):
      rvec = jnp.full((L,), 0, jnp.int32) + r
      prods = [[] for _ in range(E // L)]
      masks = [[] for _ in range(E // L)]
      for i in range(C):
        idv = plsc.load_gather(ids_v, [rvec, splats[i]])
        sv = plsc.load_gather(sc_v, [rvec, splats[i]])
        for j in range(E // L):
          rowj = plsc.load_gather(cooc_v, [idv, vals[j]])
          prods[j].append(sv * rowj)
          masks[j].append(vals[j] == idv)

      def tree(xs, op):
        while len(xs) > 1:
          xs = [op(xs[k], xs[k + 1]) for k in range(0, len(xs) - 1, 2)] + (
              [xs[-1]] if len(xs) & 1 else [])
        return xs[0]

      accs = tuple(
          jnp.where(tree(masks[j], jnp.logical_or), neg_inf,
                    tree(prods[j], jnp.add))
          for j in range(E // L))
      ids16 = plsc.load_gather(ids_v, [rvec, col8])
      s16 = plsc.load_gather(sc_v, [rvec, col8])
      return accs + (ids16, s16)

    def level1(state):
      a0, a1, a2, a3, ids16, s16 = state
      s0 = plsc.sort_key_val(a0, vals[0], descending=True)
      s1 = plsc.sort_key_val(a1, vals[1], descending=False)
      s2 = plsc.sort_key_val(a2, vals[2], descending=True)
      s3 = plsc.sort_key_val(a3, vals[3], descending=False)
      return s0, s1, s2, s3, ids16, s16

    def finish(r, state):
      (s0k, s0v), (s1k, s1v), (s2k, s2v), (s3k, s3v), ids16, s16 = state
      t01k, t01v = plsc.sort_key_val(jnp.where(mask_lo, s0k, s1k),
                                     jnp.where(mask_lo, s0v, s1v),
                                     descending=True)
      t23k, t23v = plsc.sort_key_val(jnp.where(mask_lo, s2k, s3k),
                                     jnp.where(mask_lo, s2v, s3v),
                                     descending=False)
      fk, fv = plsc.sort_key_val(jnp.where(mask_lo, t01k, t23k),
                                 jnp.where(mask_lo, t01v, t23v),
                                 descending=True)
      rvec = jnp.full((L,), 0, jnp.int32) + r
      oi_v[r] = ids16 + di
      os_v[r] = s16 + df
      plsc.store_scatter(oi_v, [rvec, ocol], fv + di, mask=mask_lo)
      plsc.store_scatter(os_v, [rvec, ocol], fk + df, mask=mask_lo)

    def stash(slot, st):
      a0, a1, a2, a3, ids16, s16 = st
      pf_v[slot, 0] = a0
      pf_v[slot, 1] = a1
      pf_v[slot, 2] = a2
      pf_v[slot, 3] = a3
      pf_v[slot, 4] = s16
      pi_v[slot] = ids16

    def unstash(slot):
      return (pf_v[slot, 0], pf_v[slot, 1], pf_v[slot, 2], pf_v[slot, 3],
              pi_v[slot], pf_v[slot, 4])

    stash(1, accum(0))

    def row_body(r, carry):
      sorted1 = level1(unstash((r + 1) & 1))
      stash(r & 1, accum(jnp.minimum(r, R - 1)))
      finish(jnp.maximum(r - 1, 0), sorted1)
      return carry

    lax.fori_loop(0, R + 1, row_body, 0)
    pltpu.sync_copy(oi_v, oi_hbm.at[pl.ds(base, R)])
    pltpu.sync_copy(os_v, os_hbm.at[pl.ds(base, R)])

  return sc_kernel


@functools.cache
def _get_sc_kernel(B: int):
  return _build_sc_kernel(B)


def kernel(candidate_scores, cooccurrence, candidate_ids, target_size):
  B, _ = candidate_ids.shape
  delta_i = jnp.asarray(target_size, jnp.int32) - K
  dvi = jnp.full((L,), delta_i, jnp.int32)
  dvf = jnp.full((L,), delta_i.astype(jnp.float32), jnp.float32)
  n_chunks = 4
  h = B // n_chunks
  f = _get_sc_kernel(h)
  outs = [f(candidate_scores[c * h:(c + 1) * h], cooccurrence,
            candidate_ids[c * h:(c + 1) * h], dvi, dvf)
          for c in range(n_chunks)]
  return (jnp.concatenate([o[0] for o in outs], axis=0),
          jnp.concatenate([o[1] for o in outs], axis=0))


# speedup vs baseline: 1.2774x; 1.0610x over previous
import functools

import jax
import jax.numpy as jnp
from jax import lax
from jax.experimental import pallas as pl
from jax.experimental.pallas import tpu as pltpu
from jax.experimental.pallas import tpu_sc as plsc

E = 64
C = 8
K = 16
L = 16
NC = 2
NS = 16
NW = NC * NS


def _build_sc_kernel(B: int):
  R = B // NW
  mesh = plsc.VectorSubcoreMesh(core_axis_name="c", subcore_axis_name="s")

  @functools.partial(
      pl.kernel,
      out_type=(
          jax.ShapeDtypeStruct((B, K), jnp.int32),
          jax.ShapeDtypeStruct((B, K), jnp.float32),
      ),
      mesh=mesh,
      compiler_params=pltpu.CompilerParams(
          needs_layout_passes=False, use_tc_tiling_on_sc=False),
      scratch_types=[
          pltpu.VMEM((E, E), jnp.float32),
          pltpu.VMEM((R, C), jnp.int32),
          pltpu.VMEM((R, C), jnp.float32),
          pltpu.VMEM((L,), jnp.int32),
          pltpu.VMEM((L,), jnp.float32),
          pltpu.VMEM((R, K), jnp.int32),
          pltpu.VMEM((R, K), jnp.float32),
          pltpu.VMEM((2, 5, L), jnp.float32),
          pltpu.VMEM((2, L), jnp.int32),
          pltpu.SemaphoreType.DMA,
      ],
  )
  def sc_kernel(scores_hbm, cooc_hbm, ids_hbm, dvi_hbm, dvf_hbm,
                oi_hbm, os_hbm,
                cooc_v, ids_v, sc_v, di_v, df_v, oi_v, os_v, pf_v, pi_v,
                dsem):
    wid = lax.axis_index("s") * NC + lax.axis_index("c")
    base = wid * R
    copies = [
        pltpu.make_async_copy(cooc_hbm, cooc_v, dsem),
        pltpu.make_async_copy(ids_hbm.at[pl.ds(base, R)], ids_v, dsem),
        pltpu.make_async_copy(scores_hbm.at[pl.ds(base, R)], sc_v, dsem),
        pltpu.make_async_copy(dvi_hbm, di_v, dsem),
        pltpu.make_async_copy(dvf_hbm, df_v, dsem),
    ]
    for cp in copies:
      cp.start()
    for cp in copies:
      cp.wait()

    lane = lax.iota(jnp.int32, L)
    mask_lo = lane < C
    neg_inf = jnp.full((L,), -jnp.inf, jnp.float32)
    vals = [lane + j * L for j in range(E // L)]
    col8 = lane & (C - 1)
    ocol = col8 + C
    splats = [jnp.full((L,), i, jnp.int32) for i in range(C)]
    di = di_v[...]
    df = df_v[...]

    def accum(r):
      rvec = jnp.full((L,), 0, jnp.int32) + r
      prods = [[] for _ in range(E // L)]
      masks = [[] for _ in range(E // L)]
      for i in range(C):
        idv = plsc.load_gather(ids_v, [rvec, splats[i]])
        sv = plsc.load_gather(sc_v, [rvec, splats[i]])
        for j in range(E // L):
          rowj = plsc.load_gather(cooc_v, [idv, vals[j]])
          prods[j].append(sv * rowj)
          masks[j].append(vals[j] == idv)

      def tree(xs, op):
        while len(xs) > 1:
          xs = [op(xs[k], xs[k + 1]) for k in range(0, len(xs) - 1, 2)] + (
              [xs[-1]] if len(xs) & 1 else [])
        return xs[0]

      accs = tuple(
          jnp.where(tree(masks[j], jnp.logical_or), neg_inf,
                    tree(prods[j], jnp.add))
          for j in range(E // L))
      ids16 = plsc.load_gather(ids_v, [rvec, col8])
      s16 = plsc.load_gather(sc_v, [rvec, col8])
      return accs + (ids16, s16)

    def level1(state):
      a0, a1, a2, a3, ids16, s16 = state
      s0 = plsc.sort_key_val(a0, vals[0], descending=True)
      s1 = plsc.sort_key_val(a1, vals[1], descending=False)
      s2 = plsc.sort_key_val(a2, vals[2], descending=True)
      s3 = plsc.sort_key_val(a3, vals[3], descending=False)
      return s0, s1, s2, s3, ids16, s16

    def finish(r, state):
      (s0k, s0v), (s1k, s1v), (s2k, s2v), (s3k, s3v), ids16, s16 = state
      t01k, t01v = plsc.sort_key_val(jnp.where(mask_lo, s0k, s1k),
                                     jnp.where(mask_lo, s0v, s1v),
                                     descending=True)
      t23k, t23v = plsc.sort_key_val(jnp.where(mask_lo, s2k, s3k),
                                     jnp.where(mask_lo, s2v, s3v),
                                     descending=False)
      fk, fv = plsc.sort_key_val(jnp.where(mask_lo, t01k, t23k),
                                 jnp.where(mask_lo, t01v, t23v),
                                 descending=True)
      rvec = jnp.full((L,), 0, jnp.int32) + r
      oi_v[r] = ids16 + di
      os_v[r] = s16 + df
      plsc.store_scatter(oi_v, [rvec, ocol], fv + di, mask=mask_lo)
      plsc.store_scatter(os_v, [rvec, ocol], fk + df, mask=mask_lo)

    def stash(slot, st):
      a0, a1, a2, a3, ids16, s16 = st
      pf_v[slot, 0] = a0
      pf_v[slot, 1] = a1
      pf_v[slot, 2] = a2
      pf_v[slot, 3] = a3
      pf_v[slot, 4] = s16
      pi_v[slot] = ids16

    def unstash(slot):
      return (pf_v[slot, 0], pf_v[slot, 1], pf_v[slot, 2], pf_v[slot, 3],
              pi_v[slot], pf_v[slot, 4])

    stash(1, accum(0))

    def row_body(r, carry):
      sorted1 = level1(unstash((r + 1) & 1))
      stash(r & 1, accum(jnp.minimum(r, R - 1)))
      finish(jnp.maximum(r - 1, 0), sorted1)
      return carry

    lax.fori_loop(0, R + 1, row_body, 0)
    pltpu.sync_copy(oi_v, oi_hbm.at[pl.ds(base, R)])
    pltpu.sync_copy(os_v, os_hbm.at[pl.ds(base, R)])

  return sc_kernel


@functools.cache
def _get_sc_kernel(B: int):
  return _build_sc_kernel(B)


def kernel(candidate_scores, cooccurrence, candidate_ids, target_size):
  B, _ = candidate_ids.shape
  delta_i = jnp.asarray(target_size, jnp.int32) - K
  dvi = jnp.full((L,), delta_i, jnp.int32)
  dvf = jnp.full((L,), delta_i.astype(jnp.float32), jnp.float32)
  n_chunks = 4
  h = B // n_chunks
  f = _get_sc_kernel(h)
  outs = [f(candidate_scores[c * h:(c + 1) * h], cooccurrence,
            candidate_ids[c * h:(c + 1) * h], dvi, dvf)
          for c in range(n_chunks)]
  return (jnp.concatenate([o[0] for o in outs], axis=0),
          jnp.concatenate([o[1] for o in outs], axis=0))
